# baseline (speedup 1.0000x reference)
import functools

import jax
import jax.numpy as jnp
from jax import lax
from jax.experimental import pallas as pl
from jax.experimental.pallas import tpu as pltpu
from jax.experimental.pallas import tpu_sc as plsc

DIM = 1024
NROW = 8192
RB = 16


@functools.partial(jax.jit, static_argnames=("total",))
def _lookup_sc(positions_flat, table, total):
    info = plsc.get_sparse_core_info()
    nc, ns = info.num_cores, info.num_subcores
    nw = nc * ns
    rows_per_w = NROW // nw
    n_vecs = total // 16
    n_chunks = rows_per_w // RB
    mesh = plsc.VectorSubcoreMesh(core_axis_name="c", subcore_axis_name="s")

    @functools.partial(
        pl.kernel,
        mesh=mesh,
        out_type=jax.ShapeDtypeStruct((total, DIM), jnp.float32),
        compiler_params=pltpu.CompilerParams(needs_layout_passes=False),
        scratch_types=[
            pltpu.VMEM((total,), jnp.int32),
            pltpu.VMEM((16 * 256,), jnp.int32),
            pltpu.VMEM((16 * 256,), jnp.int32),
            pltpu.VMEM((256,), jnp.int32),
            pltpu.VMEM((256,), jnp.int32),
            pltpu.VMEM((total + 16,), jnp.int32),
            pltpu.VMEM((RB, DIM), jnp.float32),
            pltpu.VMEM((RB, DIM), jnp.float32),
            pltpu.SemaphoreType.DMA,
            pltpu.SemaphoreType.DMA,
            pltpu.SemaphoreType.DMA,
            pltpu.SemaphoreType.DMA,
        ],
    )
    def k(table_hbm, pos_hbm, out_hbm, pos_v, hist_v, cur_v, tot_v, st_v,
          bpos, rb0, rb1, g0, g1, o0, o1):
        wid = lax.axis_index("s") * nc + lax.axis_index("c")
        lo = wid * rows_per_w
        lane = lax.iota(jnp.int32, 16)
        ones = jnp.ones((16,), jnp.int32)
        rbufs = (rb0, rb1)
        gsems = (g0, g1)
        osems = (o0, o1)

        def grab(c, b):
            return pltpu.make_async_copy(
                table_hbm.at[pl.ds(lo + c * RB, RB)], rbufs[b], gsems[b]
            )

        grab(0, 0).start()
        grab(1, 1).start()
        pltpu.sync_copy(pos_hbm, pos_v)

        def zero(i, _):
            hist_v[pl.ds(i * 16, 16)] = jnp.zeros((16,), jnp.int32)
            return 0

        lax.fori_loop(0, 256, zero, 0)

        def in_range(v):
            x = pos_v[pl.ds(v * 16, 16)]
            r = x - lo
            m = (r >= 0) & (r < rows_per_w)
            rs = jnp.where(m, r, 0)
            return m, lane * 256 + rs

        def hist_pass(v, _):
            m, idx16 = in_range(v)
            plsc.addupdate_scatter(hist_v, [idx16], ones, mask=m)
            return 0

        lax.fori_loop(0, n_vecs, hist_pass, 0)

        def totals(g, _):
            base = g * 16
            acc = jnp.zeros((16,), jnp.int32)
            for l in range(16):
                acc = acc + hist_v[pl.ds(l * 256 + base, 16)]
            tot_v[pl.ds(base, 16)] = acc
            return 0

        lax.fori_loop(0, 16, totals, 0)

        def prefix(g, carry):
            v = tot_v[pl.ds(g * 16, 16)]
            cs = plsc.cumsum(v)
            st_v[pl.ds(g * 16, 16)] = cs - v + carry
            return carry + jnp.max(cs)

        lax.fori_loop(0, 16, prefix, jnp.int32(0))

        def cursors(g, _):
            base = g * 16
            accv = st_v[pl.ds(base, 16)]
            for l in range(16):
                cur_v[pl.ds(l * 256 + base, 16)] = accv
                accv = accv + hist_v[pl.ds(l * 256 + base, 16)]
            return 0

        lax.fori_loop(0, 16, cursors, 0)

        def place(v, _):
            m, idx16 = in_range(v)
            slot = plsc.load_gather(cur_v, [idx16])
            plsc.store_scatter(bpos, [slot], v * 16 + lane, mask=m)
            plsc.addupdate_scatter(cur_v, [idx16], ones, mask=m)
            return 0

        lax.fori_loop(0, n_vecs, place, 0)

        def drain(b, n):
            def w(t, _):
                pltpu.make_async_copy(
                    rbufs[b].at[pl.ds(0, 1)], out_hbm.at[pl.ds(0, 1)], osems[b]
                ).wait()
                return 0

            lax.fori_loop(0, n, w, 0)

        def emit_pair(pc, _):
            for b in range(2):
                c = 2 * pc + b
                grab(c, b).wait()
                sv = st_v[pl.ds(c * 16, 16)]
                tv = tot_v[pl.ds(c * 16, 16)]
                for j in range(16):
                    aj = sv[j]
                    kj = tv[j]

                    def occ(t, _, aj=aj, b=b, j=j):
                        pv = bpos[pl.ds(aj + t, 16)]
                        p = pv[0]
                        pltpu.make_async_copy(
                            rbufs[b].at[pl.ds(j, 1)],
                            out_hbm.at[pl.ds(p, 1)],
                            osems[b],
                        ).start()
                        return 0

                    lax.fori_loop(0, kj, occ, 0)
                k_chunk = sv[15] + tv[15] - sv[0]

                @pl.when(c + 2 < n_chunks)
                def _(b=b, c=c, k_chunk=k_chunk):
                    drain(b, k_chunk)
                    grab(c + 2, b).start()

                @pl.when(c + 2 >= n_chunks)
                def _(b=b, k_chunk=k_chunk):
                    drain(b, k_chunk)

            return 0

        lax.fori_loop(0, n_chunks // 2, emit_pair, 0)

    return k(table, positions_flat)


def kernel(positions, table):
    b, s = positions.shape
    flat = positions.reshape(b * s).astype(jnp.int32)
    out = _lookup_sc(flat, table, b * s)
    return out.reshape(b, s, DIM)


# speedup vs baseline: 1.0106x; 1.0106x over previous
import functools

import jax
import jax.numpy as jnp
from jax import lax
from jax.experimental import pallas as pl
from jax.experimental.pallas import tpu as pltpu
from jax.experimental.pallas import tpu_sc as plsc

DIM = 1024
NROW = 8192
RB = 16


@functools.partial(jax.jit, static_argnames=("total",))
def _lookup_sc(positions_flat, table, total):
    info = plsc.get_sparse_core_info()
    nc, ns = info.num_cores, info.num_subcores
    nw = nc * ns
    rows_per_w = NROW // nw
    n_vecs = total // 16
    n_chunks = rows_per_w // RB
    mesh = plsc.VectorSubcoreMesh(core_axis_name="c", subcore_axis_name="s")

    @functools.partial(
        pl.kernel,
        mesh=mesh,
        out_type=jax.ShapeDtypeStruct((total, DIM), jnp.float32),
        compiler_params=pltpu.CompilerParams(needs_layout_passes=False),
        scratch_types=[
            pltpu.VMEM((total,), jnp.int32),
            pltpu.VMEM((16 * 256,), jnp.int32),
            pltpu.VMEM((16 * 256,), jnp.int32),
            pltpu.VMEM((256,), jnp.int32),
            pltpu.VMEM((256,), jnp.int32),
            pltpu.VMEM((total + 16,), jnp.int32),
            pltpu.VMEM((RB, DIM), jnp.float32),
            pltpu.VMEM((RB, DIM), jnp.float32),
            pltpu.SemaphoreType.DMA,
            pltpu.SemaphoreType.DMA,
            pltpu.SemaphoreType.DMA,
            pltpu.SemaphoreType.DMA,
        ],
    )
    def k(table_hbm, pos_hbm, out_hbm, pos_v, hist_v, cur_v, tot_v, st_v,
          bpos, rb0, rb1, g0, g1, o0, o1):
        wid = lax.axis_index("s") * nc + lax.axis_index("c")
        lo = wid * rows_per_w
        lane = lax.iota(jnp.int32, 16)
        ones = jnp.ones((16,), jnp.int32)
        rbufs = (rb0, rb1)
        gsems = (g0, g1)
        osems = (o0, o1)

        def grab(c, b):
            return pltpu.make_async_copy(
                table_hbm.at[pl.ds(lo + c * RB, RB)], rbufs[b], gsems[b]
            )

        grab(0, 0).start()
        grab(1, 1).start()
        pltpu.sync_copy(pos_hbm, pos_v)

        def zero(i, _):
            hist_v[pl.ds(i * 16, 16)] = jnp.zeros((16,), jnp.int32)
            return 0

        lax.fori_loop(0, 256, zero, 0)

        def in_range(v):
            x = pos_v[pl.ds(v * 16, 16)]
            r = x - lo
            m = (r >= 0) & (r < rows_per_w)
            rs = jnp.where(m, r, 0)
            return m, lane * 256 + rs

        UNROLL = 8

        def hist_pass(u, _):
            for d in range(UNROLL):
                m, idx16 = in_range(u * UNROLL + d)
                plsc.addupdate_scatter(hist_v, [idx16], ones, mask=m)
            return 0

        lax.fori_loop(0, n_vecs // UNROLL, hist_pass, 0)

        def totals(g, _):
            base = g * 16
            acc = jnp.zeros((16,), jnp.int32)
            for l in range(16):
                acc = acc + hist_v[pl.ds(l * 256 + base, 16)]
            tot_v[pl.ds(base, 16)] = acc
            return 0

        lax.fori_loop(0, 16, totals, 0)

        def prefix(g, carry):
            v = tot_v[pl.ds(g * 16, 16)]
            cs = plsc.cumsum(v)
            st_v[pl.ds(g * 16, 16)] = cs - v + carry
            return carry + jnp.max(cs)

        lax.fori_loop(0, 16, prefix, jnp.int32(0))

        def cursors(g, _):
            base = g * 16
            accv = st_v[pl.ds(base, 16)]
            for l in range(16):
                cur_v[pl.ds(l * 256 + base, 16)] = accv
                accv = accv + hist_v[pl.ds(l * 256 + base, 16)]
            return 0

        lax.fori_loop(0, 16, cursors, 0)

        def place(u, _):
            for d in range(UNROLL):
                v = u * UNROLL + d
                m, idx16 = in_range(v)
                slot = plsc.load_gather(cur_v, [idx16])
                plsc.store_scatter(bpos, [slot], v * 16 + lane, mask=m)
                plsc.addupdate_scatter(cur_v, [idx16], ones, mask=m)
            return 0

        lax.fori_loop(0, n_vecs // UNROLL, place, 0)

        def drain(b, n):
            def w(t, _):
                pltpu.make_async_copy(
                    rbufs[b].at[pl.ds(0, 1)], out_hbm.at[pl.ds(0, 1)], osems[b]
                ).wait()
                return 0

            lax.fori_loop(0, n, w, 0)

        def emit_pair(pc, _):
            for b in range(2):
                c = 2 * pc + b
                grab(c, b).wait()
                sv = st_v[pl.ds(c * 16, 16)]
                tv = tot_v[pl.ds(c * 16, 16)]
                for j in range(16):
                    aj = sv[j]
                    kj = tv[j]

                    def occ(t, _, aj=aj, b=b, j=j):
                        pv = bpos[pl.ds(aj + t, 16)]
                        p = pv[0]
                        pltpu.make_async_copy(
                            rbufs[b].at[pl.ds(j, 1)],
                            out_hbm.at[pl.ds(p, 1)],
                            osems[b],
                        ).start()
                        return 0

                    lax.fori_loop(0, kj, occ, 0)
                k_chunk = sv[15] + tv[15] - sv[0]

                @pl.when(c + 2 < n_chunks)
                def _(b=b, c=c, k_chunk=k_chunk):
                    drain(b, k_chunk)
                    grab(c + 2, b).start()

                @pl.when(c + 2 >= n_chunks)
                def _(b=b, k_chunk=k_chunk):
                    drain(b, k_chunk)

            return 0

        lax.fori_loop(0, n_chunks // 2, emit_pair, 0)

    return k(table, positions_flat)


def kernel(positions, table):
    b, s = positions.shape
    flat = positions.reshape(b * s).astype(jnp.int32)
    out = _lookup_sc(flat, table, b * s)
    return out.reshape(b, s, DIM)


# speedup vs baseline: 1.0187x; 1.0080x over previous
import functools

import jax
import jax.numpy as jnp
from jax import lax
from jax.experimental import pallas as pl
from jax.experimental.pallas import tpu as pltpu
from jax.experimental.pallas import tpu_sc as plsc

DIM = 1024
NROW = 8192
RB = 8
NSPLIT = 4


@functools.partial(jax.jit, static_argnames=("total",))
def _lookup_sc(positions_flat, table, total):
    info = plsc.get_sparse_core_info()
    nc, ns = info.num_cores, info.num_subcores
    nw = nc * ns
    rows_per_w = NROW // nw
    n_vecs = total // 16
    n_chunks = rows_per_w // RB
    mesh = plsc.VectorSubcoreMesh(core_axis_name="c", subcore_axis_name="s")

    @functools.partial(
        pl.kernel,
        mesh=mesh,
        out_type=jax.ShapeDtypeStruct((total, DIM), jnp.float32),
        compiler_params=pltpu.CompilerParams(needs_layout_passes=False),
        scratch_types=[
            pltpu.VMEM((total,), jnp.int32),
        ]
        + [pltpu.VMEM((16 * 256,), jnp.int32) for _ in range(NSPLIT)]
        + [pltpu.VMEM((16 * 256,), jnp.int32) for _ in range(NSPLIT)]
        + [
            pltpu.VMEM((272,), jnp.int32),
            pltpu.VMEM((272,), jnp.int32),
            pltpu.VMEM((total + 16,), jnp.int32),
            pltpu.VMEM((RB, DIM), jnp.float32),
            pltpu.VMEM((RB, DIM), jnp.float32),
            pltpu.SemaphoreType.DMA,
            pltpu.SemaphoreType.DMA,
            pltpu.SemaphoreType.DMA,
            pltpu.SemaphoreType.DMA,
        ],
    )
    def k(table_hbm, pos_hbm, out_hbm, pos_v, h0, h1, h2, h3, c0, c1, c2, c3,
          tot_v, st_v, bpos, rb0, rb1, g0, g1, o0, o1):
        hists = (h0, h1, h2, h3)
        curs = (c0, c1, c2, c3)
        wid = lax.axis_index("s") * nc + lax.axis_index("c")
        lo = wid * rows_per_w
        lane = lax.iota(jnp.int32, 16)
        ones = jnp.ones((16,), jnp.int32)
        rbufs = (rb0, rb1)
        gsems = (g0, g1)
        osems = (o0, o1)

        def grab(c, b):
            return pltpu.make_async_copy(
                table_hbm.at[pl.ds(lo + c * RB, RB)], rbufs[b], gsems[b]
            )

        grab(0, 0).start()
        grab(1, 1).start()
        pltpu.sync_copy(pos_hbm, pos_v)

        def zero(i, _):
            for h in hists:
                h[pl.ds(i * 16, 16)] = jnp.zeros((16,), jnp.int32)
            return 0

        lax.fori_loop(0, 256, zero, 0)

        def in_range(v):
            x = pos_v[pl.ds(v * 16, 16)]
            r = x - lo
            m = (r >= 0) & (r < rows_per_w)
            rs = jnp.where(m, r, 0)
            return m, lane * 256 + rs

        def hist_pass(u, _):
            for d in range(NSPLIT):
                m, idx16 = in_range(u * NSPLIT + d)
                plsc.addupdate_scatter(hists[d], [idx16], ones, mask=m)
            return 0

        lax.fori_loop(0, n_vecs // NSPLIT, hist_pass, 0)

        def totals(g, _):
            base = g * 16
            acc = jnp.zeros((16,), jnp.int32)
            for h in hists:
                for l in range(16):
                    acc = acc + h[pl.ds(l * 256 + base, 16)]
            tot_v[pl.ds(base, 16)] = acc
            return 0

        lax.fori_loop(0, 16, totals, 0)

        def prefix(g, carry):
            v = tot_v[pl.ds(g * 16, 16)]
            cs = plsc.cumsum(v)
            st_v[pl.ds(g * 16, 16)] = cs - v + carry
            return carry + jnp.max(cs)

        lax.fori_loop(0, 16, prefix, jnp.int32(0))

        def cursors(g, _):
            base = g * 16
            accv = st_v[pl.ds(base, 16)]
            for d in range(NSPLIT):
                for l in range(16):
                    curs[d][pl.ds(l * 256 + base, 16)] = accv
                    accv = accv + hists[d][pl.ds(l * 256 + base, 16)]
            return 0

        lax.fori_loop(0, 16, cursors, 0)

        def place(u, _):
            for d in range(NSPLIT):
                v = u * NSPLIT + d
                m, idx16 = in_range(v)
                slot = plsc.load_gather(curs[d], [idx16])
                plsc.store_scatter(bpos, [slot], v * 16 + lane, mask=m)
                plsc.addupdate_scatter(curs[d], [idx16], ones, mask=m)
            return 0

        lax.fori_loop(0, n_vecs // NSPLIT, place, 0)

        def drain(b, n):
            def w(t, _):
                pltpu.make_async_copy(
                    rbufs[b].at[pl.ds(0, 1)], out_hbm.at[pl.ds(0, 1)], osems[b]
                ).wait()
                return 0

            lax.fori_loop(0, n, w, 0)

        def emit_pair(pc, _):
            for b in range(2):
                c = 2 * pc + b
                grab(c, b).wait()
                sv = st_v[pl.ds(c * RB, 16)]
                tv = tot_v[pl.ds(c * RB, 16)]
                for j in range(RB):
                    aj = sv[j]
                    kj = tv[j]

                    def occ(t, _, aj=aj, b=b, j=j):
                        pv = bpos[pl.ds(aj + t, 16)]
                        p = pv[0]
                        pltpu.make_async_copy(
                            rbufs[b].at[pl.ds(j, 1)],
                            out_hbm.at[pl.ds(p, 1)],
                            osems[b],
                        ).start()
                        return 0

                    lax.fori_loop(0, kj, occ, 0)
                k_chunk = sv[RB - 1] + tv[RB - 1] - sv[0]

                @pl.when(c + 2 < n_chunks)
                def _(b=b, c=c, k_chunk=k_chunk):
                    drain(b, k_chunk)
                    grab(c + 2, b).start()

                @pl.when(c + 2 >= n_chunks)
                def _(b=b, k_chunk=k_chunk):
                    drain(b, k_chunk)

            return 0

        lax.fori_loop(0, n_chunks // 2, emit_pair, 0)

    return k(table, positions_flat)


def kernel(positions, table):
    b, s = positions.shape
    flat = positions.reshape(b * s).astype(jnp.int32)
    out = _lookup_sc(flat, table, b * s)
    return out.reshape(b, s, DIM)


# speedup vs baseline: 1.2763x; 1.2529x over previous
import functools

import jax
import jax.numpy as jnp
from jax import lax
from jax.experimental import pallas as pl
from jax.experimental.pallas import tpu as pltpu
from jax.experimental.pallas import tpu_sc as plsc

DIM = 1024
CHUNK = 8
NBUF = 8


@functools.partial(jax.jit, static_argnames=("total",))
def _gather_sc(positions_flat, table, total):
    info = plsc.get_sparse_core_info()
    nc, ns = info.num_cores, info.num_subcores
    nw = nc * ns
    b_per_w = total // nw
    n_chunks = b_per_w // CHUNK
    n_buf = NBUF
    n_rounds = n_chunks // n_buf
    mesh = plsc.VectorSubcoreMesh(core_axis_name="c", subcore_axis_name="s")

    @functools.partial(
        pl.kernel,
        mesh=mesh,
        out_type=jax.ShapeDtypeStruct((total, DIM), jnp.float32),
        scratch_types=[pltpu.VMEM((b_per_w,), jnp.int32)]
        + [pltpu.VMEM((CHUNK, DIM), jnp.float32) for _ in range(NBUF)]
        + [pltpu.SemaphoreType.DMA for _ in range(2 * NBUF)],
    )
    def k(table_hbm, idx_hbm, out_hbm, idx_v, *scr):
        bufs = scr[:NBUF]
        gsems = scr[NBUF : 2 * NBUF]
        osems = scr[2 * NBUF :]
        wid = lax.axis_index("s") * nc + lax.axis_index("c")
        base = wid * b_per_w
        pltpu.sync_copy(idx_hbm.at[pl.ds(base, b_per_w)], idx_v)

        def gather(i, b):
            return pltpu.make_async_copy(
                table_hbm.at[idx_v.at[pl.ds(i * CHUNK, CHUNK)]], bufs[b], gsems[b]
            )

        def put(i, b):
            return pltpu.make_async_copy(
                bufs[b], out_hbm.at[pl.ds(base + i * CHUNK, CHUNK)], osems[b]
            )

        for b in range(n_buf):
            gather(b, b).start()

        def round_body(p, _):
            i0 = p * n_buf
            for b in range(n_buf):
                i = i0 + b
                gather(i, b).wait()
                put(i, b).start()
            @pl.when(p + 1 < n_rounds)
            def _():
                for b in range(n_buf):
                    i = i0 + b
                    put(i, b).wait()
                    gather(i + n_buf, b).start()

            return 0

        lax.fori_loop(0, n_rounds, round_body, 0)
        for b in range(n_buf):
            put(n_chunks - n_buf + b, b).wait()

    return k(table, positions_flat)


def kernel(positions, table):
    b, s = positions.shape
    flat = positions.reshape(b * s).astype(jnp.int32)
    out = _gather_sc(flat, table, b * s)
    return out.reshape(b, s, DIM)
